# baseline (speedup 1.0000x reference)
import functools

import jax
import jax.numpy as jnp
from jax import lax
from jax.experimental import pallas as pl
from jax.experimental.pallas import tpu as pltpu
from jax.experimental.pallas import tpu_sc as plsc

B = 4096
HIST = 50
D = 32
NC, NS = 2, 16
NW = NC * NS

UB = B // NW
TOTAL_I = B * HIST
IB = TOTAL_I // NW
CH = 128
NCH = IB // CH


def _body(user_id_hbm, items_hbm, user_table_hbm, item_table_hbm,
          user_out_hbm, item_out_hbm,
          uidx_v, iidx_v, urow_v, irow_v, sem, sem2):
    wid = lax.axis_index("s") * NC + lax.axis_index("c")

    ubase = wid * UB
    ibase = wid * IB
    pltpu.sync_copy(user_id_hbm.at[pl.ds(ubase, UB)], uidx_v)
    pltpu.sync_copy(items_hbm.at[pl.ds(ibase, IB)], iidx_v)

    pltpu.async_copy(user_table_hbm.at[uidx_v], urow_v, sem2).wait()
    pltpu.sync_copy(urow_v, user_out_hbm.at[pl.ds(ubase, UB)])

    def chunk(c, carry):
        off = c * CH
        pltpu.async_copy(
            item_table_hbm.at[iidx_v.at[pl.ds(off, CH)]], irow_v, sem).wait()
        pltpu.sync_copy(irow_v, item_out_hbm.at[pl.ds(ibase + off, CH)])
        return carry

    lax.fori_loop(0, NCH, chunk, 0)


_grid_kernel = functools.partial(
    pl.kernel,
    out_type=(
        jax.ShapeDtypeStruct((B, D), jnp.float32),
        jax.ShapeDtypeStruct((TOTAL_I, D), jnp.float32),
    ),
    mesh=plsc.VectorSubcoreMesh(core_axis_name="c", subcore_axis_name="s",
                                num_cores=NC, num_subcores=NS),
    scratch_types=[
        pltpu.VMEM((UB,), jnp.int32),
        pltpu.VMEM((IB,), jnp.int32),
        pltpu.VMEM((UB, D), jnp.float32),
        pltpu.VMEM((CH, D), jnp.float32),
        pltpu.SemaphoreType.DMA,
        pltpu.SemaphoreType.DMA,
    ],
    compiler_params=pltpu.CompilerParams(use_tc_tiling_on_sc=False),
)(_body)


def kernel(user_id, items_ids, user_table, item_table):
    items_flat = items_ids.reshape(TOTAL_I)
    user_out, item_out = _grid_kernel(user_id, items_flat,
                                      user_table, item_table)
    return user_out, item_out.reshape(B, HIST, D)


# speedup vs baseline: 1.0365x; 1.0365x over previous
import functools

import jax
import jax.numpy as jnp
from jax import lax
from jax.experimental import pallas as pl
from jax.experimental.pallas import tpu as pltpu
from jax.experimental.pallas import tpu_sc as plsc

B = 4096
HIST = 50
D = 32
NC, NS = 2, 16
NW = NC * NS

UB = B // NW
TOTAL_I = B * HIST
IB = TOTAL_I // NW
CH = 640
NCH = IB // CH


def _body(user_id_hbm, items_hbm, user_table_hbm, item_table_hbm,
          user_out_hbm, item_out_hbm,
          uidx_v, iidx_v, urow_v, irow_v, sem, sem2):
    wid = lax.axis_index("s") * NC + lax.axis_index("c")

    ubase = wid * UB
    ibase = wid * IB
    pltpu.sync_copy(user_id_hbm.at[pl.ds(ubase, UB)], uidx_v)
    pltpu.sync_copy(items_hbm.at[pl.ds(ibase, IB)], iidx_v)

    pltpu.async_copy(user_table_hbm.at[uidx_v], urow_v, sem2).wait()
    pltpu.sync_copy(urow_v, user_out_hbm.at[pl.ds(ubase, UB)])

    def chunk(c, carry):
        off = c * CH
        pltpu.async_copy(
            item_table_hbm.at[iidx_v.at[pl.ds(off, CH)]], irow_v, sem).wait()
        pltpu.sync_copy(irow_v, item_out_hbm.at[pl.ds(ibase + off, CH)])
        return carry

    lax.fori_loop(0, NCH, chunk, 0)


_grid_kernel = functools.partial(
    pl.kernel,
    out_type=(
        jax.ShapeDtypeStruct((B, D), jnp.float32),
        jax.ShapeDtypeStruct((TOTAL_I, D), jnp.float32),
    ),
    mesh=plsc.VectorSubcoreMesh(core_axis_name="c", subcore_axis_name="s",
                                num_cores=NC, num_subcores=NS),
    scratch_types=[
        pltpu.VMEM((UB,), jnp.int32),
        pltpu.VMEM((IB,), jnp.int32),
        pltpu.VMEM((UB, D), jnp.float32),
        pltpu.VMEM((CH, D), jnp.float32),
        pltpu.SemaphoreType.DMA,
        pltpu.SemaphoreType.DMA,
    ],
    compiler_params=pltpu.CompilerParams(use_tc_tiling_on_sc=False),
)(_body)


def kernel(user_id, items_ids, user_table, item_table):
    items_flat = items_ids.reshape(TOTAL_I)
    user_out, item_out = _grid_kernel(user_id, items_flat,
                                      user_table, item_table)
    return user_out, item_out.reshape(B, HIST, D)


# speedup vs baseline: 1.2553x; 1.2111x over previous
import functools

import jax
import jax.numpy as jnp
from jax import lax
from jax.experimental import pallas as pl
from jax.experimental.pallas import tpu as pltpu
from jax.experimental.pallas import tpu_sc as plsc

B = 4096
HIST = 50
D = 32
NC, NS = 2, 16
NW = NC * NS

UB = B // NW
TOTAL_I = B * HIST
BW = B // NW
IB = BW * HIST
CHB = 64
CH = CHB * HIST
NCH = BW // CHB


def _body(user_id_hbm, items_hbm, user_table_hbm, item_table_hbm,
          user_out_hbm, item_out_hbm,
          uidx_v, iidx_v, urow_v, irow_v, sem, sem2):
    wid = lax.axis_index("s") * NC + lax.axis_index("c")

    ubase = wid * UB
    b0 = wid * BW
    pltpu.sync_copy(user_id_hbm.at[pl.ds(ubase, UB)], uidx_v)
    pltpu.sync_copy(items_hbm.at[pl.ds(b0 * HIST, IB)], iidx_v)

    pltpu.async_copy(user_table_hbm.at[uidx_v], urow_v, sem2).wait()
    pltpu.sync_copy(urow_v, user_out_hbm.at[pl.ds(ubase, UB)])

    for c in range(NCH):
        pltpu.async_copy(
            item_table_hbm.at[iidx_v.at[pl.ds(c * CH, CH)]], irow_v, sem).wait()
        for j in range(CHB):
            pltpu.sync_copy(irow_v.at[pl.ds(j * HIST, HIST)],
                            item_out_hbm.at[b0 + c * CHB + j])


_grid_kernel = functools.partial(
    pl.kernel,
    out_type=(
        jax.ShapeDtypeStruct((B, D), jnp.float32),
        jax.ShapeDtypeStruct((B, HIST, D), jnp.float32),
    ),
    mesh=plsc.VectorSubcoreMesh(core_axis_name="c", subcore_axis_name="s",
                                num_cores=NC, num_subcores=NS),
    scratch_types=[
        pltpu.VMEM((UB,), jnp.int32),
        pltpu.VMEM((IB,), jnp.int32),
        pltpu.VMEM((UB, D), jnp.float32),
        pltpu.VMEM((CH, D), jnp.float32),
        pltpu.SemaphoreType.DMA,
        pltpu.SemaphoreType.DMA,
    ],
    compiler_params=pltpu.CompilerParams(use_tc_tiling_on_sc=False),
)(_body)


def kernel(user_id, items_ids, user_table, item_table):
    items_flat = items_ids.reshape(TOTAL_I)
    user_out, item_out = _grid_kernel(user_id, items_flat,
                                      user_table, item_table)
    return user_out, item_out
